# baseline (speedup 1.0000x reference)
import functools

import numpy as np
import jax
import jax.numpy as jnp
from jax import lax
from jax.experimental import pallas as pl
from jax.experimental.pallas import tpu as pltpu
from jax.experimental.pallas import tpu_sc as plsc

GRID = 200
XMIN, XMAX = -5.0, 5.0
DX = (XMAX - XMIN) / GRID
LOG_DXDY = float(np.log(DX) + np.log(DX))

NC, NS, L = 2, 16, 16
NW = NC * NS
B = 16384
BPW = B // NW
GCH = 128
NG = BPW // GCH

GP = 256
BLK = 2048
NBLK = B // BLK
BIGI = 1 << 30


def _grid_tables():
    xg = jnp.linspace(XMIN, XMAX, GRID)
    g2 = xg * xg
    bg = xg.astype(jnp.bfloat16).astype(jnp.float32)
    g2p = jnp.concatenate([g2, jnp.full((GP - GRID,), 1e30, jnp.float32)])
    bgp = jnp.concatenate([bg, jnp.zeros((GP - GRID,), jnp.float32)])
    return g2p.reshape(GP, 1), bgp.reshape(GP, 1)


def _first_true_idx(mask, iota):
    return jnp.min(jnp.where(mask, iota, BIGI), axis=0, keepdims=True)


def _tc_ids(tb_t, g2p, bgp):
    def body(tb_ref, g2_ref, bg_ref, o_ref):
        x = tb_ref[0, :][None, :]
        y = tb_ref[1, :][None, :]
        xb = x.astype(jnp.bfloat16).astype(jnp.float32)
        yb = y.astype(jnp.bfloat16).astype(jnp.float32)
        an = x * x + y * y
        g2 = g2_ref[...]
        bg = bg_ref[...]
        iota = lax.broadcasted_iota(jnp.int32, (GP, BLK), 0)
        hx = g2 - 2.0 * (xb * bg)
        hy = g2 - 2.0 * (yb * bg)
        mx = jnp.min(hx, axis=0, keepdims=True)
        my = jnp.min(hy, axis=0, keepdims=True)
        kxm = _first_true_idx(hx == mx, iota)
        kym = _first_true_idx(hy == my, iota)
        clamp = (an + mx + my) <= 0.0
        kx0 = _first_true_idx(hx <= -(an + my), iota)
        hxs = jnp.sum(jnp.where(iota == kx0, hx, 0.0), axis=0,
                      keepdims=True)
        ky0 = _first_true_idx(hy <= -(an + hxs), iota)
        kx = jnp.where(clamp, kx0, kxm)
        ky = jnp.where(clamp, ky0, kym)
        o_ref[...] = jnp.reshape(kx * GRID + ky, (BLK,))

    return pl.pallas_call(
        body,
        grid=(NBLK,),
        in_specs=[
            pl.BlockSpec((2, BLK), lambda i: (0, i)),
            pl.BlockSpec((GP, 1), lambda i: (0, 0)),
            pl.BlockSpec((GP, 1), lambda i: (0, 0)),
        ],
        out_specs=pl.BlockSpec((BLK,), lambda i: (i,)),
        out_shape=jax.ShapeDtypeStruct((B,), jnp.int32),
    )(tb_t, g2p, bgp)


def _sc_gather_partials(ids, E):
    mesh = plsc.VectorSubcoreMesh(core_axis_name="c", subcore_axis_name="s")

    @functools.partial(
        pl.kernel,
        out_type=jax.ShapeDtypeStruct((NW, L), jnp.float32),
        mesh=mesh,
        scratch_types=[
            pltpu.VMEM((NG, GCH), jnp.int32),
            pltpu.VMEM((BPW,), jnp.float32),
            pltpu.VMEM((L,), jnp.float32),
            pltpu.SemaphoreType.DMA,
            pltpu.SemaphoreType.DMA,
        ],
    )
    def k(ids_hbm, e_hbm, out_hbm, idxv, valv, accv, sem_i, sem_g):
        wid = lax.axis_index("s") * NC + lax.axis_index("c")
        base = wid * BPW
        idx_copies = [
            pltpu.async_copy(ids_hbm.at[pl.ds(base + c * GCH, GCH)],
                             idxv.at[c], sem_i)
            for c in range(NG)
        ]
        accv[...] = jnp.zeros((L,), jnp.float32)
        gathers = []
        for c in range(NG):
            idx_copies[c].wait()
            gathers.append(
                pltpu.async_copy(e_hbm.at[idxv.at[c]],
                                 valv.at[pl.ds(c * GCH, GCH)], sem_g))
        for c in range(NG):
            gathers[c].wait()
            for i in range(GCH // L):
                accv[...] = accv[...] + valv[pl.ds(c * GCH + i * L, L)]
        pltpu.sync_copy(accv, out_hbm.at[wid])

    return k(ids, E)


def _tc_combine(e2d, partials):
    def body(e_ref, p_ref, o_ref):
        se = jnp.sum(jnp.exp(-e_ref[...]))
        mean = jnp.sum(p_ref[...]) * (1.0 / B)
        val = -mean - jnp.log(se) - LOG_DXDY
        o_ref[...] = jnp.reshape(val, (1, 1))

    return pl.pallas_call(
        body,
        out_shape=jax.ShapeDtypeStruct((1, 1), jnp.float32),
    )(e2d, partials)


def kernel(train_batch, E):
    tb_t = train_batch.T
    g2p, bgp = _grid_tables()
    ids = _tc_ids(tb_t, g2p, bgp)
    partials = _sc_gather_partials(ids, E)
    out = _tc_combine(E.reshape(GRID, GRID), partials)
    return out[0, 0]


# speedup vs baseline: 1.1189x; 1.1189x over previous
import functools

import numpy as np
import jax
import jax.numpy as jnp
from jax import lax
from jax.experimental import pallas as pl
from jax.experimental.pallas import tpu as pltpu
from jax.experimental.pallas import tpu_sc as plsc

GRID = 200
XMIN, XMAX = -5.0, 5.0
DX = (XMAX - XMIN) / GRID
LOG_DXDY = float(np.log(DX) + np.log(DX))

NC, NS, L = 2, 16, 16
NW = NC * NS
B = 16384
BPW = B // NW
GCH = 128
NG = BPW // GCH

GP = 256
BLK = 2048
NBLK = B // BLK
BIGI = 1 << 30


def _grid_tables():
    xg = jnp.linspace(XMIN, XMAX, GRID)
    g2 = xg * xg
    bg = xg.astype(jnp.bfloat16).astype(jnp.float32)
    g2p = jnp.concatenate([g2, jnp.full((GP - GRID,), 1e30, jnp.float32)])
    bgp = jnp.concatenate([bg, jnp.zeros((GP - GRID,), jnp.float32)])
    return g2p.reshape(GP, 1), bgp.reshape(GP, 1)


def _first_true_idx(mask, iota):
    return jnp.min(jnp.where(mask, iota, BIGI), axis=0, keepdims=True)


def _tc_ids(tb_t, g2p, bgp):
    def body(tb_ref, g2_ref, bg_ref, o_ref):
        x = tb_ref[0, :][None, :]
        y = tb_ref[1, :][None, :]
        xb = x.astype(jnp.bfloat16).astype(jnp.float32)
        yb = y.astype(jnp.bfloat16).astype(jnp.float32)
        an = x * x + y * y
        g2 = g2_ref[...]
        bg = bg_ref[...]
        iota = lax.broadcasted_iota(jnp.int32, (GP, BLK), 0)
        hx = g2 - 2.0 * (xb * bg)
        hy = g2 - 2.0 * (yb * bg)
        mx = jnp.min(hx, axis=0, keepdims=True)
        my = jnp.min(hy, axis=0, keepdims=True)
        kx = _first_true_idx(hx <= jnp.maximum(mx, -(an + my)), iota)
        hxs = jnp.sum(jnp.where(iota == kx, hx, 0.0), axis=0,
                      keepdims=True)
        ky = _first_true_idx(hy <= jnp.maximum(my, -(an + hxs)), iota)
        o_ref[...] = jnp.reshape(kx * GRID + ky, (BLK,))

    return pl.pallas_call(
        body,
        grid=(NBLK,),
        in_specs=[
            pl.BlockSpec((2, BLK), lambda i: (0, i)),
            pl.BlockSpec((GP, 1), lambda i: (0, 0)),
            pl.BlockSpec((GP, 1), lambda i: (0, 0)),
        ],
        out_specs=pl.BlockSpec((BLK,), lambda i: (i,)),
        out_shape=jax.ShapeDtypeStruct((B,), jnp.int32),
    )(tb_t, g2p, bgp)


def _sc_gather_partials(ids, E):
    mesh = plsc.VectorSubcoreMesh(core_axis_name="c", subcore_axis_name="s")

    @functools.partial(
        pl.kernel,
        out_type=jax.ShapeDtypeStruct((NW, L), jnp.float32),
        mesh=mesh,
        scratch_types=[
            pltpu.VMEM((NG, GCH), jnp.int32),
            pltpu.VMEM((BPW,), jnp.float32),
            pltpu.VMEM((L,), jnp.float32),
            pltpu.SemaphoreType.DMA,
            pltpu.SemaphoreType.DMA,
        ],
    )
    def k(ids_hbm, e_hbm, out_hbm, idxv, valv, accv, sem_i, sem_g):
        wid = lax.axis_index("s") * NC + lax.axis_index("c")
        base = wid * BPW
        idx_copies = [
            pltpu.async_copy(ids_hbm.at[pl.ds(base + c * GCH, GCH)],
                             idxv.at[c], sem_i)
            for c in range(NG)
        ]
        accv[...] = jnp.zeros((L,), jnp.float32)
        gathers = []
        for c in range(NG):
            idx_copies[c].wait()
            gathers.append(
                pltpu.async_copy(e_hbm.at[idxv.at[c]],
                                 valv.at[pl.ds(c * GCH, GCH)], sem_g))
        for c in range(NG):
            gathers[c].wait()
            for i in range(GCH // L):
                accv[...] = accv[...] + valv[pl.ds(c * GCH + i * L, L)]
        pltpu.sync_copy(accv, out_hbm.at[wid])

    return k(ids, E)


def _tc_combine(e2d, partials):
    def body(e_ref, p_ref, o_ref):
        se = jnp.sum(jnp.exp(-e_ref[...]))
        mean = jnp.sum(p_ref[...]) * (1.0 / B)
        val = -mean - jnp.log(se) - LOG_DXDY
        o_ref[...] = jnp.reshape(val, (1, 1))

    return pl.pallas_call(
        body,
        out_shape=jax.ShapeDtypeStruct((1, 1), jnp.float32),
    )(e2d, partials)


def kernel(train_batch, E):
    tb_t = train_batch.T
    g2p, bgp = _grid_tables()
    ids = _tc_ids(tb_t, g2p, bgp)
    partials = _sc_gather_partials(ids, E)
    out = _tc_combine(E.reshape(GRID, GRID), partials)
    return out[0, 0]
